# baseline (speedup 1.0000x reference)
import jax
import jax.numpy as jnp
from jax import lax
from jax.experimental import pallas as pl
from jax.experimental.pallas import tpu as pltpu
from jax.experimental.pallas import tpu_sc as plsc

_BS = 16
_H = 32
_W = 32
_F = 256
_NCHAN = 2 * _F
_CPW = _NCHAN // 32
_BLK = _CPW * _H * _W


def _sc_body(tab_hbm, out_hbm, tab_v, blk_v, sem):
    wid = lax.axis_index("s") * 2 + lax.axis_index("c")
    c0 = wid * _CPW
    pltpu.sync_copy(tab_hbm.at[pl.ds(c0 * _H, _CPW * _H)], tab_v)

    is_top = wid < 16

    def chan_top(cc, carry):
        v0 = tab_v[pl.ds(cc * _H, 16)]
        v1 = tab_v[pl.ds(cc * _H + 16, 16)]
        base = cc * (_H * _W)
        for y in range(_H):
            blk_v[pl.ds(base + y * _W, 16)] = v0
            blk_v[pl.ds(base + y * _W + 16, 16)] = v1
        return carry

    def chan_bot(cc, carry):
        base = cc * (_H * _W)
        vv0 = tab_v[pl.ds(cc * _H, 16)]
        vv1 = tab_v[pl.ds(cc * _H + 16, 16)]
        for y in range(16):
            v = jnp.full((16,), vv0[y], jnp.float32)
            blk_v[pl.ds(base + y * _W, 16)] = v
            blk_v[pl.ds(base + y * _W + 16, 16)] = v
        for y in range(16):
            v = jnp.full((16,), vv1[y], jnp.float32)
            blk_v[pl.ds(base + (y + 16) * _W, 16)] = v
            blk_v[pl.ds(base + (y + 16) * _W + 16, 16)] = v
        return carry

    @pl.when(is_top)
    def _():
        lax.fori_loop(0, _CPW, chan_top, 0)

    @pl.when(jnp.logical_not(is_top))
    def _():
        lax.fori_loop(0, _CPW, chan_bot, 0)

    copies = [
        pltpu.async_copy(blk_v, out_hbm.at[b, pl.ds(c0 * _H * _W, _BLK)], sem)
        for b in range(_BS)
    ]
    for c in copies:
        c.wait()


def kernel(mask, row_embed, col_embed):
    bs, h, w = mask.shape
    f = row_embed.shape[1]
    tab = jnp.concatenate([col_embed[:w], row_embed[:h]], axis=1).T.reshape(-1)
    mesh = plsc.VectorSubcoreMesh(core_axis_name="c", subcore_axis_name="s")
    run = pl.kernel(
        _sc_body,
        out_type=jax.ShapeDtypeStruct((bs, 2 * f * h * w), jnp.float32),
        mesh=mesh,
        scratch_types=[
            pltpu.VMEM((_CPW * _H,), jnp.float32),
            pltpu.VMEM((_BLK,), jnp.float32),
            pltpu.SemaphoreType.DMA,
        ],
    )
    return run(tab).reshape(bs, 2 * f, h, w)


# speedup vs baseline: 1.4161x; 1.4161x over previous
import jax
import jax.numpy as jnp
from jax import lax
from jax.experimental import pallas as pl
from jax.experimental.pallas import tpu as pltpu
from jax.experimental.pallas import tpu_sc as plsc

_BS = 16
_H = 32
_W = 32
_F = 256
_NCHAN = 2 * _F
_CPW = _NCHAN // 32
_BLK = _CPW * _H * _W


def _sc_body(tab_hbm, out_hbm, tab_v, blk_v, sem):
    wid = lax.axis_index("s") * 2 + lax.axis_index("c")
    c0 = wid * _CPW
    pltpu.sync_copy(tab_hbm.at[pl.ds(c0 * _H, _CPW * _H)], tab_v)

    is_top = wid < 16

    def chan_top(cc, carry):
        v0 = tab_v[pl.ds(cc * _H, 16)]
        v1 = tab_v[pl.ds(cc * _H + 16, 16)]
        for y in range(_H):
            blk_v[cc, y, pl.ds(0, 16)] = v0
            blk_v[cc, y, pl.ds(16, 16)] = v1
        return carry

    def chan_bot(cc, carry):
        vv0 = tab_v[pl.ds(cc * _H, 16)]
        vv1 = tab_v[pl.ds(cc * _H + 16, 16)]
        for y in range(16):
            v = jnp.full((16,), vv0[y], jnp.float32)
            blk_v[cc, y, pl.ds(0, 16)] = v
            blk_v[cc, y, pl.ds(16, 16)] = v
        for y in range(16):
            v = jnp.full((16,), vv1[y], jnp.float32)
            blk_v[cc, y + 16, pl.ds(0, 16)] = v
            blk_v[cc, y + 16, pl.ds(16, 16)] = v
        return carry

    @pl.when(is_top)
    def _():
        lax.fori_loop(0, _CPW, chan_top, 0)

    @pl.when(jnp.logical_not(is_top))
    def _():
        lax.fori_loop(0, _CPW, chan_bot, 0)

    copies = [
        pltpu.async_copy(blk_v, out_hbm.at[b, pl.ds(c0, _CPW)], sem)
        for b in range(_BS)
    ]
    for c in copies:
        c.wait()


def kernel(mask, row_embed, col_embed):
    bs, h, w = mask.shape
    f = row_embed.shape[1]
    tab = jnp.concatenate([col_embed[:w], row_embed[:h]], axis=1).T.reshape(-1)
    mesh = plsc.VectorSubcoreMesh(core_axis_name="c", subcore_axis_name="s")
    run = pl.kernel(
        _sc_body,
        out_type=jax.ShapeDtypeStruct((bs, 2 * f, h, w), jnp.float32),
        mesh=mesh,
        scratch_types=[
            pltpu.VMEM((_CPW * _H,), jnp.float32),
            pltpu.VMEM((_CPW, _H, _W), jnp.float32),
            pltpu.SemaphoreType.DMA,
        ],
    )
    return run(tab)


# speedup vs baseline: 5.8432x; 4.1263x over previous
import jax
import jax.numpy as jnp
from jax import lax
from jax.experimental import pallas as pl
from jax.experimental.pallas import tpu as pltpu
from jax.experimental.pallas import tpu_sc as plsc

_BS = 16
_H = 32
_W = 32
_F = 256


def _sc_body(col_hbm, row_hbm, out_hbm, blk_v, row_v, sem):
    y = lax.axis_index("s") * 2 + lax.axis_index("c")
    pltpu.sync_copy(col_hbm.at[pl.ds(0, _W)], blk_v.at[:, pl.ds(0, _F)])
    pltpu.sync_copy(row_hbm.at[y], row_v)
    for j in range(_F // 16):
        v = row_v[pl.ds(j * 16, 16)]
        for x in range(_W):
            blk_v[x, pl.ds(_F + j * 16, 16)] = v

    copies = [
        pltpu.async_copy(blk_v, out_hbm.at[b, y], sem) for b in range(_BS)
    ]
    for c in copies:
        c.wait()


def kernel(mask, row_embed, col_embed):
    bs, h, w = mask.shape
    f = row_embed.shape[1]
    mesh = plsc.VectorSubcoreMesh(core_axis_name="c", subcore_axis_name="s")
    run = pl.kernel(
        _sc_body,
        out_type=jax.ShapeDtypeStruct((bs, h, w, 2 * f), jnp.float32),
        mesh=mesh,
        scratch_types=[
            pltpu.VMEM((_W, 2 * _F), jnp.float32),
            pltpu.VMEM((_F,), jnp.float32),
            pltpu.SemaphoreType.DMA,
        ],
    )
    out = run(col_embed, row_embed)
    return jnp.transpose(out, (0, 3, 1, 2))
